# baseline (device time: 174259 ns/iter reference)
import jax
import jax.numpy as jnp
from jax import lax
from jax.experimental import pallas as pl
from jax.experimental.pallas import tpu as pltpu

T = 4096
D = 2048
V_LOC = 8192
NZ = 4
GROUP = T // 4
CHA = GROUP // NZ
CHB = CHA // 2


def _fused_gather_allreduce(local_ids, mask, E):

    def body(ids_ref, mask_ref, e_ref, out_ref, gbuf, gsems,
             comm_ref, a_send, a_recv, credit_sem, b_send, b_recv):
        x = lax.axis_index("x")
        y = lax.axis_index("y")
        z = lax.axis_index("z")
        p = 2 * x + y
        px = 2 * (1 - x) + y
        py = 2 * x + (1 - y)
        right = (z + 1) % NZ
        left = (z - 1) % NZ
        base = p * GROUP
        xn = (1 - x, y, z)
        yn = (x, 1 - y, z)

        def row_copy(c, i, row):
            return pltpu.make_async_copy(
                e_ref.at[pl.ds(row, 1)],
                gbuf.at[pl.ds(c * CHA + i, 1)],
                gsems.at[c],
            )

        def _hit(c, i):
            row = ids_ref[c * CHA + i]
            return row, jnp.logical_and(row >= 0, row < V_LOC)

        def gather_issue(c):
            def f(i, _):
                row, ok = _hit(c, i)

                @pl.when(ok)
                def _():
                    row_copy(c, i, row).start()

                return 0
            lax.fori_loop(0, CHA, f, 0)

        def gather_finish(c):
            def f(i, _):
                row, ok = _hit(c, i)

                @pl.when(ok)
                def _():
                    row_copy(c, i, row).wait()

                return 0
            lax.fori_loop(0, CHA, f, 0)
            out_ref[pl.ds(base + c * CHA, CHA)] = jnp.where(
                mask_ref[pl.ds(c * CHA, CHA)] > 0,
                gbuf[pl.ds(c * CHA, CHA)],
                0.0,
            ).astype(jnp.bfloat16)

        def ring_step(s):
            slot = s % 2
            if s < NZ - 1:
                cs = (z - s) % NZ
                cr = (z - s - 1) % NZ
            else:
                t = s - (NZ - 1)
                cs = (z + 1 - t) % NZ
                cr = (z - t) % NZ
            if s >= 2:
                pl.semaphore_wait(credit_sem, 1)
            rdma = pltpu.make_async_remote_copy(
                src_ref=out_ref.at[pl.ds(base + cs * CHA, CHA)],
                dst_ref=comm_ref.at[slot],
                send_sem=a_send.at[slot],
                recv_sem=a_recv.at[slot],
                device_id=(x, y, right),
                device_id_type=pl.DeviceIdType.MESH,
            )
            rdma.start()
            return rdma, slot, cr

        def ring_finish(s, rdma, slot, cr):
            rdma.wait()
            if s < 2 * (NZ - 1) - 2:
                pl.semaphore_signal(
                    credit_sem,
                    inc=1,
                    device_id=(x, y, left),
                    device_id_type=pl.DeviceIdType.MESH,
                )
            if s < NZ - 1:
                out_ref[pl.ds(base + cr * CHA, CHA)] = (
                    out_ref[pl.ds(base + cr * CHA, CHA)] + comm_ref[slot]
                )
            else:
                out_ref[pl.ds(base + cr * CHA, CHA)] = comm_ref[slot]

        c_order = [z, (z - 1) % NZ, (z - 2) % NZ, (z + 1) % NZ]
        for c in c_order:
            gather_issue(c)
        gather_finish(c_order[0])

        for s in range(NZ - 1):
            started = ring_step(s)
            gather_finish(c_order[s + 1])
            ring_finish(s, *started)

        def xfer(g, off, target, idx):
            sl = pl.ds(g * GROUP + off, CHB)
            return pltpu.make_async_remote_copy(
                src_ref=out_ref.at[sl],
                dst_ref=out_ref.at[sl],
                send_sem=b_send.at[idx],
                recv_sem=b_recv.at[idx],
                device_id=target,
                device_id_type=pl.DeviceIdType.MESH,
            )

        def b1_start(c, k):
            b1x = xfer(p, c * CHA, xn, 6 * k)
            b1y = xfer(p, c * CHA + CHB, yn, 6 * k + 1)
            b1x.start()
            b1y.start()
            return b1x, b1y

        def b2_start(c, k):
            rs = [
                xfer(p, c * CHA, yn, 6 * k + 2),
                xfer(px, c * CHA, yn, 6 * k + 3),
                xfer(p, c * CHA + CHB, xn, 6 * k + 4),
                xfer(py, c * CHA + CHB, xn, 6 * k + 5),
            ]
            for r in rs:
                r.start()
            return rs

        own = (z + 1) % NZ
        b1_pend = b1_start(own, 0)
        prev_c, prev_k = own, 0
        b2_pend = []
        for t in range(NZ - 1):
            s = (NZ - 1) + t
            started = ring_step(s)
            b1_pend[0].wait()
            b1_pend[1].wait()
            b2_pend.extend(b2_start(prev_c, prev_k))
            ring_finish(s, *started)
            cr = started[2]
            b1_pend = b1_start(cr, t + 1)
            prev_c, prev_k = cr, t + 1
        b1_pend[0].wait()
        b1_pend[1].wait()
        b2_pend.extend(b2_start(prev_c, prev_k))
        for r in b2_pend:
            r.wait()

    return pl.pallas_call(
        body,
        out_shape=jax.ShapeDtypeStruct((T, D), jnp.bfloat16),
        in_specs=[
            pl.BlockSpec(memory_space=pltpu.SMEM),
            pl.BlockSpec(memory_space=pltpu.VMEM),
            pl.BlockSpec(memory_space=pl.ANY),
        ],
        out_specs=pl.BlockSpec(memory_space=pltpu.VMEM),
        scratch_shapes=[
            pltpu.VMEM((GROUP, D), jnp.float32),
            pltpu.SemaphoreType.DMA((NZ,)),
            pltpu.VMEM((2, CHA, D), jnp.bfloat16),
            pltpu.SemaphoreType.DMA((2,)),
            pltpu.SemaphoreType.DMA((2,)),
            pltpu.SemaphoreType.REGULAR,
            pltpu.SemaphoreType.DMA((24,)),
            pltpu.SemaphoreType.DMA((24,)),
        ],
        compiler_params=pltpu.CompilerParams(vmem_limit_bytes=60 * 1024 * 1024),
    )(local_ids, mask, E)


def kernel(ids, E):
    x = lax.axis_index("x")
    y = lax.axis_index("y")
    z = lax.axis_index("z")
    p = 2 * x + y
    ids_p = lax.dynamic_slice(ids, (p * GROUP,), (GROUP,))
    local = (ids_p - z * V_LOC).astype(jnp.int32)
    ok = (local >= 0) & (local < V_LOC)
    mask = ok.astype(jnp.float32)[:, None]
    return _fused_gather_allreduce(local, mask, E)


# device time: 159890 ns/iter; 1.0899x vs baseline; 1.0899x over previous
import jax
import jax.numpy as jnp
from jax import lax
from jax.experimental import pallas as pl
from jax.experimental.pallas import tpu as pltpu

T = 4096
D = 2048
V_LOC = 8192
NZ = 4
GROUP = T // 4
CHA = GROUP // NZ
CHB = CHA // 2


def _fused_gather_allreduce(local_ids, order, offs, mask, E):

    def body(ids_ref, order_ref, offs_ref, mask_ref, e_ref, out_ref, gbuf, gsems,
             comm_ref, a_send, a_recv, credit_sem, b_send, b_recv):
        x = lax.axis_index("x")
        y = lax.axis_index("y")
        z = lax.axis_index("z")
        p = 2 * x + y
        px = 2 * (1 - x) + y
        py = 2 * x + (1 - y)
        right = (z + 1) % NZ
        left = (z - 1) % NZ
        base = p * GROUP
        xn = (1 - x, y, z)
        yn = (x, 1 - y, z)

        def row_copy(c, i):
            return pltpu.make_async_copy(
                e_ref.at[pl.ds(ids_ref[i], 1)],
                gbuf.at[pl.ds(i, 1)],
                gsems.at[c],
            )

        def gather_issue(c):
            def f(j, _):
                row_copy(c, order_ref[j]).start()
                return 0
            lax.fori_loop(offs_ref[c], offs_ref[c + 1], f, 0)

        def gather_finish(c):
            def f(j, _):
                row_copy(c, order_ref[j]).wait()
                return 0
            lax.fori_loop(offs_ref[c], offs_ref[c + 1], f, 0)
            out_ref[pl.ds(base + c * CHA, CHA)] = jnp.where(
                mask_ref[pl.ds(c * CHA, CHA)] > 0,
                gbuf[pl.ds(c * CHA, CHA)],
                0.0,
            ).astype(jnp.bfloat16)

        def ring_step(s):
            slot = s % 2
            if s < NZ - 1:
                cs = (z - s) % NZ
                cr = (z - s - 1) % NZ
            else:
                t = s - (NZ - 1)
                cs = (z + 1 - t) % NZ
                cr = (z - t) % NZ
            if s >= 2:
                pl.semaphore_wait(credit_sem, 1)
            rdma = pltpu.make_async_remote_copy(
                src_ref=out_ref.at[pl.ds(base + cs * CHA, CHA)],
                dst_ref=comm_ref.at[slot],
                send_sem=a_send.at[slot],
                recv_sem=a_recv.at[slot],
                device_id=(x, y, right),
                device_id_type=pl.DeviceIdType.MESH,
            )
            rdma.start()
            return rdma, slot, cr

        def ring_finish(s, rdma, slot, cr):
            rdma.wait()
            if s < 2 * (NZ - 1) - 2:
                pl.semaphore_signal(
                    credit_sem,
                    inc=1,
                    device_id=(x, y, left),
                    device_id_type=pl.DeviceIdType.MESH,
                )
            if s < NZ - 1:
                out_ref[pl.ds(base + cr * CHA, CHA)] = (
                    out_ref[pl.ds(base + cr * CHA, CHA)] + comm_ref[slot]
                )
            else:
                out_ref[pl.ds(base + cr * CHA, CHA)] = comm_ref[slot]

        c_order = [z, (z - 1) % NZ, (z - 2) % NZ, (z + 1) % NZ]
        for c in c_order:
            gather_issue(c)
        gather_finish(c_order[0])

        for s in range(NZ - 1):
            started = ring_step(s)
            gather_finish(c_order[s + 1])
            ring_finish(s, *started)

        def xfer(g, off, target, idx):
            sl = pl.ds(g * GROUP + off, CHB)
            return pltpu.make_async_remote_copy(
                src_ref=out_ref.at[sl],
                dst_ref=out_ref.at[sl],
                send_sem=b_send.at[idx],
                recv_sem=b_recv.at[idx],
                device_id=target,
                device_id_type=pl.DeviceIdType.MESH,
            )

        def b1_start(c, k):
            b1x = xfer(p, c * CHA, xn, 6 * k)
            b1y = xfer(p, c * CHA + CHB, yn, 6 * k + 1)
            b1x.start()
            b1y.start()
            return b1x, b1y

        def b2_start(c, k):
            rs = [
                xfer(p, c * CHA, yn, 6 * k + 2),
                xfer(px, c * CHA, yn, 6 * k + 3),
                xfer(p, c * CHA + CHB, xn, 6 * k + 4),
                xfer(py, c * CHA + CHB, xn, 6 * k + 5),
            ]
            for r in rs:
                r.start()
            return rs

        own = (z + 1) % NZ
        b1_pend = b1_start(own, 0)
        prev_c, prev_k = own, 0
        b2_pend = []
        for t in range(NZ - 1):
            s = (NZ - 1) + t
            started = ring_step(s)
            b1_pend[0].wait()
            b1_pend[1].wait()
            b2_pend.extend(b2_start(prev_c, prev_k))
            ring_finish(s, *started)
            cr = started[2]
            b1_pend = b1_start(cr, t + 1)
            prev_c, prev_k = cr, t + 1
        b1_pend[0].wait()
        b1_pend[1].wait()
        b2_pend.extend(b2_start(prev_c, prev_k))
        for r in b2_pend:
            r.wait()

    return pl.pallas_call(
        body,
        out_shape=jax.ShapeDtypeStruct((T, D), jnp.bfloat16),
        in_specs=[
            pl.BlockSpec(memory_space=pltpu.SMEM),
            pl.BlockSpec(memory_space=pltpu.SMEM),
            pl.BlockSpec(memory_space=pltpu.SMEM),
            pl.BlockSpec(memory_space=pltpu.VMEM),
            pl.BlockSpec(memory_space=pl.ANY),
        ],
        out_specs=pl.BlockSpec(memory_space=pltpu.VMEM),
        scratch_shapes=[
            pltpu.VMEM((GROUP, D), jnp.float32),
            pltpu.SemaphoreType.DMA((NZ,)),
            pltpu.VMEM((2, CHA, D), jnp.bfloat16),
            pltpu.SemaphoreType.DMA((2,)),
            pltpu.SemaphoreType.DMA((2,)),
            pltpu.SemaphoreType.REGULAR,
            pltpu.SemaphoreType.DMA((24,)),
            pltpu.SemaphoreType.DMA((24,)),
        ],
        compiler_params=pltpu.CompilerParams(vmem_limit_bytes=60 * 1024 * 1024),
    )(local_ids, order, offs, mask, E)


def kernel(ids, E):
    x = lax.axis_index("x")
    y = lax.axis_index("y")
    z = lax.axis_index("z")
    p = 2 * x + y
    ids_p = lax.dynamic_slice(ids, (p * GROUP,), (GROUP,))
    local = (ids_p - z * V_LOC).astype(jnp.int32)
    ok = (local >= 0) & (local < V_LOC)
    mask = ok.astype(jnp.float32)[:, None]
    pos = jnp.arange(GROUP, dtype=jnp.int32)
    order = jnp.argsort(jnp.where(ok, pos, pos + GROUP)).astype(jnp.int32)
    cnt = jnp.sum(ok.reshape(NZ, CHA), axis=1, dtype=jnp.int32)
    offs = jnp.concatenate(
        [jnp.zeros((1,), jnp.int32), jnp.cumsum(cnt, dtype=jnp.int32)]
    )
    return _fused_gather_allreduce(local, order, offs, mask, E)


# device time: 155618 ns/iter; 1.1198x vs baseline; 1.0275x over previous
import jax
import jax.numpy as jnp
from jax import lax
from jax.experimental import pallas as pl
from jax.experimental.pallas import tpu as pltpu

T = 4096
D = 2048
V_LOC = 8192
NZ = 4
GROUP = T // 4
CHA = GROUP // NZ
CHB = CHA // 2


def _fused_gather_allreduce(local_ids, order, offs, mask, E):

    def body(ids_ref, order_ref, offs_ref, mask_ref, e_ref, out_ref, acc_ref,
             gbuf, gsems, osems, comm_ref, a_send, a_recv, credit_sem,
             b_send, b_recv):
        x = lax.axis_index("x")
        y = lax.axis_index("y")
        z = lax.axis_index("z")
        p = 2 * x + y
        px = 2 * (1 - x) + y
        py = 2 * x + (1 - y)
        right = (z + 1) % NZ
        left = (z - 1) % NZ
        base = p * GROUP
        xn = (1 - x, y, z)
        yn = (x, 1 - y, z)

        def row_copy(c, i):
            return pltpu.make_async_copy(
                e_ref.at[pl.ds(ids_ref[i], 1)],
                gbuf.at[pl.ds(i, 1)],
                gsems.at[c],
            )

        def gather_issue(c):
            def f(j, _):
                row_copy(c, order_ref[j]).start()
                return 0
            lax.fori_loop(offs_ref[c], offs_ref[c + 1], f, 0)

        def gather_finish(c):
            def f(j, _):
                row_copy(c, order_ref[j]).wait()
                return 0
            lax.fori_loop(offs_ref[c], offs_ref[c + 1], f, 0)
            acc_ref[pl.ds(base + c * CHA, CHA)] = jnp.where(
                mask_ref[pl.ds(c * CHA, CHA)] > 0,
                gbuf[pl.ds(c * CHA, CHA)],
                0.0,
            ).astype(jnp.bfloat16)

        def ring_step(s):
            slot = s % 2
            if s < NZ - 1:
                cs = (z - s) % NZ
                cr = (z - s - 1) % NZ
            else:
                t = s - (NZ - 1)
                cs = (z + 1 - t) % NZ
                cr = (z - t) % NZ
            if s >= 2:
                pl.semaphore_wait(credit_sem, 1)
            rdma = pltpu.make_async_remote_copy(
                src_ref=acc_ref.at[pl.ds(base + cs * CHA, CHA)],
                dst_ref=comm_ref.at[slot],
                send_sem=a_send.at[slot],
                recv_sem=a_recv.at[slot],
                device_id=(x, y, right),
                device_id_type=pl.DeviceIdType.MESH,
            )
            rdma.start()
            return rdma, slot, cr

        def ring_finish(s, rdma, slot, cr):
            rdma.wait()
            if s < 2 * (NZ - 1) - 2:
                pl.semaphore_signal(
                    credit_sem,
                    inc=1,
                    device_id=(x, y, left),
                    device_id_type=pl.DeviceIdType.MESH,
                )
            if s < NZ - 1:
                acc_ref[pl.ds(base + cr * CHA, CHA)] = (
                    acc_ref[pl.ds(base + cr * CHA, CHA)] + comm_ref[slot]
                )
            else:
                acc_ref[pl.ds(base + cr * CHA, CHA)] = comm_ref[slot]

        c_order = [z, (z - 1) % NZ, (z - 2) % NZ, (z + 1) % NZ]
        for c in c_order:
            gather_issue(c)
        gather_finish(c_order[0])

        for s in range(NZ - 1):
            started = ring_step(s)
            gather_finish(c_order[s + 1])
            ring_finish(s, *started)

        def xfer(g, off, target, idx):
            sl = pl.ds(g * GROUP + off, CHB)
            return pltpu.make_async_remote_copy(
                src_ref=acc_ref.at[sl],
                dst_ref=acc_ref.at[sl],
                send_sem=b_send.at[idx],
                recv_sem=b_recv.at[idx],
                device_id=target,
                device_id_type=pl.DeviceIdType.MESH,
            )

        def b1_start(c, k):
            b1x = xfer(p, c * CHA, xn, 6 * k)
            b1y = xfer(p, c * CHA + CHB, yn, 6 * k + 1)
            b1x.start()
            b1y.start()
            return b1x, b1y

        def b2_start(c, k):
            rs = [
                xfer(p, c * CHA, yn, 6 * k + 2),
                xfer(px, c * CHA, yn, 6 * k + 3),
                xfer(p, c * CHA + CHB, xn, 6 * k + 4),
                xfer(py, c * CHA + CHB, xn, 6 * k + 5),
            ]
            for r in rs:
                r.start()
            return rs

        out_pend = []

        def out_copy(g, c):
            sl = pl.ds(g * GROUP + c * CHA, CHA)
            cp = pltpu.make_async_copy(
                acc_ref.at[sl], out_ref.at[sl], osems.at[len(out_pend)]
            )
            cp.start()
            out_pend.append(cp)

        own = (z + 1) % NZ
        out_copy(p, own)
        b1_pend = b1_start(own, 0)
        chunk_of_k = [own]
        b2_by_k = []
        for t in range(NZ - 1):
            s = (NZ - 1) + t
            started = ring_step(s)
            b1_pend[0].wait()
            b1_pend[1].wait()
            b2_by_k.append(b2_start(chunk_of_k[t], t))
            if t >= 1:
                for r in b2_by_k[t - 1]:
                    r.wait()
                for g in (px, py, 3 - p):
                    out_copy(g, chunk_of_k[t - 1])
            ring_finish(s, *started)
            cr = started[2]
            out_copy(p, cr)
            b1_pend = b1_start(cr, t + 1)
            chunk_of_k.append(cr)
        b1_pend[0].wait()
        b1_pend[1].wait()
        b2_by_k.append(b2_start(chunk_of_k[NZ - 1], NZ - 1))
        for k in (NZ - 2, NZ - 1):
            for r in b2_by_k[k]:
                r.wait()
            for g in (px, py, 3 - p):
                out_copy(g, chunk_of_k[k])
        for cp in out_pend:
            cp.wait()

    return pl.pallas_call(
        body,
        out_shape=jax.ShapeDtypeStruct((T, D), jnp.bfloat16),
        in_specs=[
            pl.BlockSpec(memory_space=pltpu.SMEM),
            pl.BlockSpec(memory_space=pltpu.SMEM),
            pl.BlockSpec(memory_space=pltpu.SMEM),
            pl.BlockSpec(memory_space=pltpu.VMEM),
            pl.BlockSpec(memory_space=pl.ANY),
        ],
        out_specs=pl.BlockSpec(memory_space=pl.ANY),
        scratch_shapes=[
            pltpu.VMEM((T, D), jnp.bfloat16),
            pltpu.VMEM((GROUP, D), jnp.float32),
            pltpu.SemaphoreType.DMA((NZ,)),
            pltpu.SemaphoreType.DMA((16,)),
            pltpu.VMEM((2, CHA, D), jnp.bfloat16),
            pltpu.SemaphoreType.DMA((2,)),
            pltpu.SemaphoreType.DMA((2,)),
            pltpu.SemaphoreType.REGULAR,
            pltpu.SemaphoreType.DMA((24,)),
            pltpu.SemaphoreType.DMA((24,)),
        ],
        compiler_params=pltpu.CompilerParams(vmem_limit_bytes=60 * 1024 * 1024),
    )(local_ids, order, offs, mask, E)


def kernel(ids, E):
    x = lax.axis_index("x")
    y = lax.axis_index("y")
    z = lax.axis_index("z")
    p = 2 * x + y
    ids_p = lax.dynamic_slice(ids, (p * GROUP,), (GROUP,))
    local = (ids_p - z * V_LOC).astype(jnp.int32)
    ok = (local >= 0) & (local < V_LOC)
    mask = ok.astype(jnp.float32)[:, None]
    pos = jnp.arange(GROUP, dtype=jnp.int32)
    order = jnp.argsort(jnp.where(ok, pos, pos + GROUP)).astype(jnp.int32)
    cnt = jnp.sum(ok.reshape(NZ, CHA), axis=1, dtype=jnp.int32)
    offs = jnp.concatenate(
        [jnp.zeros((1,), jnp.int32), jnp.cumsum(cnt, dtype=jnp.int32)]
    )
    return _fused_gather_allreduce(local, order, offs, mask, E)


# device time: 150126 ns/iter; 1.1608x vs baseline; 1.0366x over previous
import jax
import jax.numpy as jnp
from jax import lax
from jax.experimental import pallas as pl
from jax.experimental.pallas import tpu as pltpu

T = 4096
D = 2048
V_LOC = 8192
NZ = 4
GROUP = T // 4
CHA = GROUP // NZ
CHB = CHA // 2


def _fused_gather_allreduce(local_ids, order, offs, mask, E):

    def body(ids_ref, order_ref, offs_ref, mask_ref, e_ref, out_ref, acc_ref,
             gbuf, gsems, osems, comm_ref, a_send, a_recv, credit_sem,
             ag_send, ag_recv, b_send, b_recv):
        x = lax.axis_index("x")
        y = lax.axis_index("y")
        z = lax.axis_index("z")
        p = 2 * x + y
        px = 2 * (1 - x) + y
        py = 2 * x + (1 - y)
        right = (z + 1) % NZ
        left = (z - 1) % NZ
        base = p * GROUP
        xn = (1 - x, y, z)
        yn = (x, 1 - y, z)

        barrier = pltpu.get_barrier_semaphore()
        for nb in [(x, y, right), (x, y, left), xn, yn]:
            pl.semaphore_signal(
                barrier, inc=1, device_id=nb,
                device_id_type=pl.DeviceIdType.MESH,
            )
        pl.semaphore_wait(barrier, 4)

        def row_copy(c, i):
            return pltpu.make_async_copy(
                e_ref.at[pl.ds(ids_ref[i], 1)],
                gbuf.at[pl.ds(i, 1)],
                gsems.at[c],
            )

        def gather_issue(c):
            def f(j, _):
                row_copy(c, order_ref[j]).start()
                return 0
            lax.fori_loop(offs_ref[c], offs_ref[c + 1], f, 0)

        def gather_finish(c):
            def f(j, _):
                row_copy(c, order_ref[j]).wait()
                return 0
            lax.fori_loop(offs_ref[c], offs_ref[c + 1], f, 0)
            acc_ref[pl.ds(base + c * CHA, CHA)] = jnp.where(
                mask_ref[pl.ds(c * CHA, CHA)] > 0,
                gbuf[pl.ds(c * CHA, CHA)],
                0.0,
            ).astype(jnp.bfloat16)

        def rs_step(s):
            slot = s % 2
            cs = (z - s) % NZ
            cr = (z - s - 1) % NZ
            if s == 2:
                pl.semaphore_wait(credit_sem, 1)
            rdma = pltpu.make_async_remote_copy(
                src_ref=acc_ref.at[pl.ds(base + cs * CHA, CHA)],
                dst_ref=comm_ref.at[slot],
                send_sem=a_send.at[slot],
                recv_sem=a_recv.at[slot],
                device_id=(x, y, right),
                device_id_type=pl.DeviceIdType.MESH,
            )
            rdma.start()
            return rdma, slot, cr

        def rs_finish(s, rdma, slot, cr):
            rdma.wait()
            acc_ref[pl.ds(base + cr * CHA, CHA)] = (
                acc_ref[pl.ds(base + cr * CHA, CHA)] + comm_ref[slot]
            )
            if s == 0:
                pl.semaphore_signal(
                    credit_sem,
                    inc=1,
                    device_id=(x, y, left),
                    device_id_type=pl.DeviceIdType.MESH,
                )

        def ag_step(t):
            cs = (z + 1 - t) % NZ
            cr = (z - t) % NZ
            sl = pl.ds(base + cs * CHA, CHA)
            rdma = pltpu.make_async_remote_copy(
                src_ref=acc_ref.at[sl],
                dst_ref=acc_ref.at[sl],
                send_sem=ag_send.at[t],
                recv_sem=ag_recv.at[t],
                device_id=(x, y, right),
                device_id_type=pl.DeviceIdType.MESH,
            )
            rdma.start()
            return rdma, cr

        c_order = [z, (z - 1) % NZ, (z - 2) % NZ, (z + 1) % NZ]
        for c in c_order:
            gather_issue(c)
        gather_finish(c_order[0])

        for s in range(NZ - 1):
            started = rs_step(s)
            gather_finish(c_order[s + 1])
            rs_finish(s, *started)

        def xfer(g, off, target, idx):
            sl = pl.ds(g * GROUP + off, CHB)
            return pltpu.make_async_remote_copy(
                src_ref=acc_ref.at[sl],
                dst_ref=acc_ref.at[sl],
                send_sem=b_send.at[idx],
                recv_sem=b_recv.at[idx],
                device_id=target,
                device_id_type=pl.DeviceIdType.MESH,
            )

        def b1_start(c, k):
            b1x = xfer(p, c * CHA, xn, 6 * k)
            b1y = xfer(p, c * CHA + CHB, yn, 6 * k + 1)
            b1x.start()
            b1y.start()
            return b1x, b1y

        def b2_start(c, k):
            rs = [
                xfer(p, c * CHA, yn, 6 * k + 2),
                xfer(px, c * CHA, yn, 6 * k + 3),
                xfer(p, c * CHA + CHB, xn, 6 * k + 4),
                xfer(py, c * CHA + CHB, xn, 6 * k + 5),
            ]
            for r in rs:
                r.start()
            return rs

        out_pend = []

        def out_copy(g, c):
            sl = pl.ds(g * GROUP + c * CHA, CHA)
            cp = pltpu.make_async_copy(
                acc_ref.at[sl], out_ref.at[sl], osems.at[len(out_pend)]
            )
            cp.start()
            out_pend.append(cp)

        own = (z + 1) % NZ
        out_copy(p, own)
        b1_pend = b1_start(own, 0)
        chunk_of_k = [own]
        b2_by_k = []
        for t in range(NZ - 1):
            rdma, cr = ag_step(t)
            b1_pend[0].wait()
            b1_pend[1].wait()
            b2_by_k.append(b2_start(chunk_of_k[t], t))
            if t >= 1:
                for r in b2_by_k[t - 1]:
                    r.wait()
                for g in (px, py, 3 - p):
                    out_copy(g, chunk_of_k[t - 1])
            rdma.wait()
            out_copy(p, cr)
            b1_pend = b1_start(cr, t + 1)
            chunk_of_k.append(cr)
        b1_pend[0].wait()
        b1_pend[1].wait()
        b2_by_k.append(b2_start(chunk_of_k[NZ - 1], NZ - 1))
        for k in (NZ - 2, NZ - 1):
            for r in b2_by_k[k]:
                r.wait()
            for g in (px, py, 3 - p):
                out_copy(g, chunk_of_k[k])
        for cp in out_pend:
            cp.wait()

    return pl.pallas_call(
        body,
        out_shape=jax.ShapeDtypeStruct((T, D), jnp.bfloat16),
        in_specs=[
            pl.BlockSpec(memory_space=pltpu.SMEM),
            pl.BlockSpec(memory_space=pltpu.SMEM),
            pl.BlockSpec(memory_space=pltpu.SMEM),
            pl.BlockSpec(memory_space=pltpu.VMEM),
            pl.BlockSpec(memory_space=pl.ANY),
        ],
        out_specs=pl.BlockSpec(memory_space=pl.ANY),
        scratch_shapes=[
            pltpu.VMEM((T, D), jnp.bfloat16),
            pltpu.VMEM((GROUP, D), jnp.float32),
            pltpu.SemaphoreType.DMA((NZ,)),
            pltpu.SemaphoreType.DMA((16,)),
            pltpu.VMEM((2, CHA, D), jnp.bfloat16),
            pltpu.SemaphoreType.DMA((2,)),
            pltpu.SemaphoreType.DMA((2,)),
            pltpu.SemaphoreType.REGULAR,
            pltpu.SemaphoreType.DMA((NZ - 1,)),
            pltpu.SemaphoreType.DMA((NZ - 1,)),
            pltpu.SemaphoreType.DMA((24,)),
            pltpu.SemaphoreType.DMA((24,)),
        ],
        compiler_params=pltpu.CompilerParams(
            collective_id=0, vmem_limit_bytes=60 * 1024 * 1024
        ),
    )(local_ids, order, offs, mask, E)


def kernel(ids, E):
    x = lax.axis_index("x")
    y = lax.axis_index("y")
    z = lax.axis_index("z")
    p = 2 * x + y
    ids_p = lax.dynamic_slice(ids, (p * GROUP,), (GROUP,))
    local = (ids_p - z * V_LOC).astype(jnp.int32)
    ok = (local >= 0) & (local < V_LOC)
    mask = ok.astype(jnp.float32)[:, None]
    pos = jnp.arange(GROUP, dtype=jnp.int32)
    order = jnp.argsort(jnp.where(ok, pos, pos + GROUP)).astype(jnp.int32)
    cnt = jnp.sum(ok.reshape(NZ, CHA), axis=1, dtype=jnp.int32)
    offs = jnp.concatenate(
        [jnp.zeros((1,), jnp.int32), jnp.cumsum(cnt, dtype=jnp.int32)]
    )
    return _fused_gather_allreduce(local, order, offs, mask, E)


# device time: 147890 ns/iter; 1.1783x vs baseline; 1.0151x over previous
import jax
import jax.numpy as jnp
from jax import lax
from jax.experimental import pallas as pl
from jax.experimental.pallas import tpu as pltpu

T = 4096
D = 2048
V_LOC = 8192
NZ = 4
GROUP = T // 4
CHA = GROUP // NZ
CHB = CHA // 2


def _fused_gather_allreduce(local_ids, order, offs, mask, E):

    def body(ids_ref, order_ref, offs_ref, mask_ref, e_ref, out_ref, acc_ref,
             gbuf, gsems, osems, comm_ref, a_send, a_recv, credit_sem,
             ag_send, ag_recv, b_send, b_recv):
        x = lax.axis_index("x")
        y = lax.axis_index("y")
        z = lax.axis_index("z")
        p = 2 * x + y
        px = 2 * (1 - x) + y
        py = 2 * x + (1 - y)
        right = (z + 1) % NZ
        left = (z - 1) % NZ
        base = p * GROUP
        xn = (1 - x, y, z)
        yn = (x, 1 - y, z)

        barrier = pltpu.get_barrier_semaphore()
        for nb in [(x, y, right), (x, y, left), xn, yn]:
            pl.semaphore_signal(
                barrier, inc=1, device_id=nb,
                device_id_type=pl.DeviceIdType.MESH,
            )
        pl.semaphore_wait(barrier, 4)

        def row_copy(c, i):
            return pltpu.make_async_copy(
                e_ref.at[pl.ds(ids_ref[i], 1)],
                gbuf.at[pl.ds(i, 1)],
                gsems.at[c],
            )

        def gather_issue(c):
            def f(j, _):
                row_copy(c, order_ref[j]).start()
                return 0
            lax.fori_loop(offs_ref[c], offs_ref[c + 1], f, 0)

        def gather_finish(c):
            def f(j, _):
                row_copy(c, order_ref[j]).wait()
                return 0
            lax.fori_loop(offs_ref[c], offs_ref[c + 1], f, 0)
            acc_ref[pl.ds(base + c * CHA, CHA)] = jnp.where(
                mask_ref[pl.ds(c * CHA, CHA)] > 0,
                gbuf[pl.ds(c * CHA, CHA)],
                0.0,
            ).astype(jnp.bfloat16)

        def rs_half_start(s, h):
            slot = s % 2
            cs = (z - s) % NZ
            rdma = pltpu.make_async_remote_copy(
                src_ref=acc_ref.at[pl.ds(base + cs * CHA + h * CHB, CHB)],
                dst_ref=comm_ref.at[slot, pl.ds(h * CHB, CHB)],
                send_sem=a_send.at[2 * slot + h],
                recv_sem=a_recv.at[2 * slot + h],
                device_id=(x, y, right),
                device_id_type=pl.DeviceIdType.MESH,
            )
            rdma.start()
            return rdma

        def rs_half_add(s, h, rdma):
            slot = s % 2
            cr = (z - s - 1) % NZ
            rdma.wait()
            sl = pl.ds(base + cr * CHA + h * CHB, CHB)
            acc_ref[sl] = acc_ref[sl] + comm_ref[slot, pl.ds(h * CHB, CHB)]

        def ag_step(t):
            cs = (z + 1 - t) % NZ
            cr = (z - t) % NZ
            sl = pl.ds(base + cs * CHA, CHA)
            rdma = pltpu.make_async_remote_copy(
                src_ref=acc_ref.at[sl],
                dst_ref=acc_ref.at[sl],
                send_sem=ag_send.at[t],
                recv_sem=ag_recv.at[t],
                device_id=(x, y, right),
                device_id_type=pl.DeviceIdType.MESH,
            )
            rdma.start()
            return rdma, cr

        c_order = [z, (z - 1) % NZ, (z - 2) % NZ, (z + 1) % NZ]
        for c in c_order:
            gather_issue(c)
        gather_finish(c_order[0])

        d0 = rs_half_start(0, 0)
        d1 = rs_half_start(0, 1)
        for s in range(NZ - 1):
            gather_finish(c_order[s + 1])
            rs_half_add(s, 0, d0)
            if s + 1 < NZ - 1:
                if s + 1 == 2:
                    pl.semaphore_wait(credit_sem, 1)
                d0 = rs_half_start(s + 1, 0)
            rs_half_add(s, 1, d1)
            if s == 0:
                pl.semaphore_signal(
                    credit_sem,
                    inc=1,
                    device_id=(x, y, left),
                    device_id_type=pl.DeviceIdType.MESH,
                )
            if s + 1 < NZ - 1:
                d1 = rs_half_start(s + 1, 1)

        def xfer(g, off, target, idx):
            sl = pl.ds(g * GROUP + off, CHB)
            return pltpu.make_async_remote_copy(
                src_ref=acc_ref.at[sl],
                dst_ref=acc_ref.at[sl],
                send_sem=b_send.at[idx],
                recv_sem=b_recv.at[idx],
                device_id=target,
                device_id_type=pl.DeviceIdType.MESH,
            )

        def b1_start(c, k):
            b1x = xfer(p, c * CHA, xn, 6 * k)
            b1y = xfer(p, c * CHA + CHB, yn, 6 * k + 1)
            b1x.start()
            b1y.start()
            return b1x, b1y

        def b2_start(c, k):
            rs = [
                xfer(p, c * CHA, yn, 6 * k + 2),
                xfer(px, c * CHA, yn, 6 * k + 3),
                xfer(p, c * CHA + CHB, xn, 6 * k + 4),
                xfer(py, c * CHA + CHB, xn, 6 * k + 5),
            ]
            for r in rs:
                r.start()
            return rs

        out_pend = []

        def out_copy(g, c):
            sl = pl.ds(g * GROUP + c * CHA, CHA)
            cp = pltpu.make_async_copy(
                acc_ref.at[sl], out_ref.at[sl], osems.at[len(out_pend)]
            )
            cp.start()
            out_pend.append(cp)

        own = (z + 1) % NZ
        out_copy(p, own)
        b1_pend = b1_start(own, 0)
        chunk_of_k = [own]
        b2_by_k = []
        for t in range(NZ - 1):
            rdma, cr = ag_step(t)
            b1_pend[0].wait()
            b1_pend[1].wait()
            b2_by_k.append(b2_start(chunk_of_k[t], t))
            if t >= 1:
                for r in b2_by_k[t - 1]:
                    r.wait()
                for g in (px, py, 3 - p):
                    out_copy(g, chunk_of_k[t - 1])
            rdma.wait()
            out_copy(p, cr)
            b1_pend = b1_start(cr, t + 1)
            chunk_of_k.append(cr)
        b1_pend[0].wait()
        b1_pend[1].wait()
        b2_by_k.append(b2_start(chunk_of_k[NZ - 1], NZ - 1))
        for k in (NZ - 2, NZ - 1):
            for r in b2_by_k[k]:
                r.wait()
            for g in (px, py, 3 - p):
                out_copy(g, chunk_of_k[k])
        for cp in out_pend:
            cp.wait()

    return pl.pallas_call(
        body,
        out_shape=jax.ShapeDtypeStruct((T, D), jnp.bfloat16),
        in_specs=[
            pl.BlockSpec(memory_space=pltpu.SMEM),
            pl.BlockSpec(memory_space=pltpu.SMEM),
            pl.BlockSpec(memory_space=pltpu.SMEM),
            pl.BlockSpec(memory_space=pltpu.VMEM),
            pl.BlockSpec(memory_space=pl.ANY),
        ],
        out_specs=pl.BlockSpec(memory_space=pl.ANY),
        scratch_shapes=[
            pltpu.VMEM((T, D), jnp.bfloat16),
            pltpu.VMEM((GROUP, D), jnp.float32),
            pltpu.SemaphoreType.DMA((NZ,)),
            pltpu.SemaphoreType.DMA((16,)),
            pltpu.VMEM((2, CHA, D), jnp.bfloat16),
            pltpu.SemaphoreType.DMA((4,)),
            pltpu.SemaphoreType.DMA((4,)),
            pltpu.SemaphoreType.REGULAR,
            pltpu.SemaphoreType.DMA((NZ - 1,)),
            pltpu.SemaphoreType.DMA((NZ - 1,)),
            pltpu.SemaphoreType.DMA((24,)),
            pltpu.SemaphoreType.DMA((24,)),
        ],
        compiler_params=pltpu.CompilerParams(
            collective_id=0, vmem_limit_bytes=60 * 1024 * 1024
        ),
    )(local_ids, order, offs, mask, E)


def kernel(ids, E):
    x = lax.axis_index("x")
    y = lax.axis_index("y")
    z = lax.axis_index("z")
    p = 2 * x + y
    ids_p = lax.dynamic_slice(ids, (p * GROUP,), (GROUP,))
    local = (ids_p - z * V_LOC).astype(jnp.int32)
    ok = (local >= 0) & (local < V_LOC)
    mask = ok.astype(jnp.float32)[:, None]
    pos = jnp.arange(GROUP, dtype=jnp.int32)
    order = jnp.argsort(jnp.where(ok, pos, pos + GROUP)).astype(jnp.int32)
    cnt = jnp.sum(ok.reshape(NZ, CHA), axis=1, dtype=jnp.int32)
    offs = jnp.concatenate(
        [jnp.zeros((1,), jnp.int32), jnp.cumsum(cnt, dtype=jnp.int32)]
    )
    return _fused_gather_allreduce(local, order, offs, mask, E)
